# baseline (speedup 1.0000x reference)
import functools

import jax
import jax.numpy as jnp
import numpy as np
from jax import lax
from jax.experimental import pallas as pl
from jax.experimental.pallas import tpu as pltpu
from jax.experimental.pallas import tpu_sc as plsc

N = 10000
E = 320000
D = 128

_INFO = plsc.get_sparse_core_info()
NC = _INFO.num_cores
NS = _INFO.num_subcores
NW = NC * NS
ROWS_PER_TILE = 320
NPAD = NW * ROWS_PER_TILE
CHUNK = 4000
NCHUNKS = E // CHUNK
GSUB = 128
LANES = 16
FB = D // LANES
DUMMY = ROWS_PER_TILE


def _segmax_body(p_hbm, src_hbm, dst_hbm, lotab_hbm, agg_hbm,
                 acc, srcb, dstb, csrc, cdst, grows, lov_v, sem):
    wid = lax.axis_index("s") * NC + lax.axis_index("c")
    lo = wid * ROWS_PER_TILE

    zf = jnp.zeros((LANES,), jnp.float32)
    zi = jnp.zeros((LANES,), jnp.int32)
    one = jnp.ones((LANES,), jnp.int32)
    dumv = jnp.full((LANES,), DUMMY, jnp.int32)

    pltpu.sync_copy(lotab_hbm.at[wid], lov_v)
    lov = lov_v[...]

    def zero_row(r, carry):
        for j in range(FB):
            acc[r, pl.ds(j * LANES, LANES)] = zf
        return carry
    lax.fori_loop(0, ROWS_PER_TILE + 1, zero_row, 0)

    def chunk_body(k, carry):
        off = k * CHUNK
        pltpu.sync_copy(src_hbm.at[pl.ds(off, CHUNK)], srcb)
        pltpu.sync_copy(dst_hbm.at[pl.ds(off, CHUNK)], dstb)

        def zero_c(i, c):
            csrc[pl.ds(i * LANES, LANES)] = zi
            cdst[pl.ds(i * LANES, LANES)] = dumv
            return c
        lax.fori_loop(0, CHUNK // LANES, zero_c, 0)

        def grp(i, wp):
            s16 = srcb[pl.ds(i * LANES, LANES)]
            d16 = dstb[pl.ds(i * LANES, LANES)]
            dl = d16 - lov
            m = (dl >= 0) & (dl < ROWS_PER_TILE)
            mi = jnp.where(m, one, zi)
            pos = wp + plsc.cumsum(mi) - 1
            plsc.store_scatter(csrc, [pos], s16, mask=m)
            plsc.store_scatter(cdst, [pos], dl, mask=m)
            return wp + plsc.all_reduce_population_count(m)
        wpv = lax.fori_loop(0, CHUNK // LANES, grp, jnp.zeros((LANES,), jnp.int32))
        cnt = jnp.max(wpv)

        ngs = (cnt + GSUB - 1) // GSUB

        def gsub(g, c):
            base = g * GSUB
            pltpu.async_copy(p_hbm.at[csrc.at[pl.ds(base, GSUB)]], grows, sem).wait()
            ne = jnp.minimum(GSUB, cnt - base)
            ngrp = (ne + LANES - 1) // LANES

            def egrp(i, c2):
                e0 = i * LANES
                dl16 = cdst[pl.ds(base + e0, LANES)]
                for l in range(LANES):
                    d = dl16[l]
                    for j in range(FB):
                        sl = pl.ds(j * LANES, LANES)
                        acc[d, sl] = jnp.maximum(acc[d, sl], grows[e0 + l, sl])
                return c2
            lax.fori_loop(0, ngrp, egrp, 0)
            return c
        lax.fori_loop(0, ngs, gsub, 0)
        return carry

    lax.fori_loop(0, NCHUNKS, chunk_body, 0)
    pltpu.sync_copy(acc.at[pl.ds(0, ROWS_PER_TILE)],
                    agg_hbm.at[pl.ds(lo, ROWS_PER_TILE)])


_segmax = functools.partial(
    pl.kernel,
    out_type=jax.ShapeDtypeStruct((NPAD, D), jnp.float32),
    mesh=plsc.VectorSubcoreMesh(core_axis_name="c", subcore_axis_name="s"),
    compiler_params=pltpu.CompilerParams(needs_layout_passes=False),
    scratch_types=[
        pltpu.VMEM((ROWS_PER_TILE + 1, D), jnp.float32),
        pltpu.VMEM((CHUNK,), jnp.int32),
        pltpu.VMEM((CHUNK,), jnp.int32),
        pltpu.VMEM((CHUNK,), jnp.int32),
        pltpu.VMEM((CHUNK,), jnp.int32),
        pltpu.VMEM((GSUB, D), jnp.float32),
        pltpu.VMEM((LANES,), jnp.int32),
        pltpu.SemaphoreType.DMA,
    ],
)(_segmax_body)

_LOTAB = np.tile(
    (np.arange(NW, dtype=np.int32) * ROWS_PER_TILE)[:, None], (1, LANES))


BLK = 1000


def _pool_tc_body(x_ref, w_ref, b_ref, p_ref):
    p_ref[...] = jnp.maximum(
        jnp.dot(x_ref[...], w_ref[...], preferred_element_type=jnp.float32)
        + b_ref[...], 0.0)


def _combine_pool_tc_body(x_ref, agg_ref, ws_ref, wn_ref, b_ref, wp_ref,
                          bp_ref, h_ref, p_ref):
    h = jnp.maximum(
        jnp.dot(x_ref[...], ws_ref[...], preferred_element_type=jnp.float32)
        + jnp.dot(agg_ref[...], wn_ref[...], preferred_element_type=jnp.float32)
        + b_ref[...], 0.0)
    h_ref[...] = h
    p_ref[...] = jnp.maximum(
        jnp.dot(h, wp_ref[...], preferred_element_type=jnp.float32)
        + bp_ref[...], 0.0)


def _combine_tc_body(x_ref, agg_ref, ws_ref, wn_ref, b_ref, h_ref):
    h_ref[...] = jnp.maximum(
        jnp.dot(x_ref[...], ws_ref[...], preferred_element_type=jnp.float32)
        + jnp.dot(agg_ref[...], wn_ref[...], preferred_element_type=jnp.float32)
        + b_ref[...], 0.0)


def _row_spec():
    return pl.BlockSpec((BLK, D), lambda i: (i, 0))


def _full_spec():
    return pl.BlockSpec((D, D), lambda i: (0, 0))


def _bias_spec():
    return pl.BlockSpec((1, D), lambda i: (0, 0))


def _pool_tc(x, w, b):
    return pl.pallas_call(
        _pool_tc_body,
        grid=(N // BLK,),
        in_specs=[_row_spec(), _full_spec(), _bias_spec()],
        out_specs=_row_spec(),
        out_shape=jax.ShapeDtypeStruct((N, D), jnp.float32),
    )(x, w, b.reshape(1, D))


def _combine_pool_tc(x, agg, ws, wn, b, wp, bp):
    return pl.pallas_call(
        _combine_pool_tc_body,
        grid=(N // BLK,),
        in_specs=[_row_spec(), _row_spec(), _full_spec(), _full_spec(),
                  _bias_spec(), _full_spec(), _bias_spec()],
        out_specs=[_row_spec(), _row_spec()],
        out_shape=[jax.ShapeDtypeStruct((N, D), jnp.float32),
                   jax.ShapeDtypeStruct((N, D), jnp.float32)],
    )(x, agg, ws, wn, b.reshape(1, D), wp, bp.reshape(1, D))


def _combine_tc(x, agg, ws, wn, b):
    return pl.pallas_call(
        _combine_tc_body,
        grid=(N // BLK,),
        in_specs=[_row_spec(), _row_spec(), _full_spec(), _full_spec(),
                  _bias_spec()],
        out_specs=_row_spec(),
        out_shape=jax.ShapeDtypeStruct((N, D), jnp.float32),
    )(x, agg, ws, wn, b.reshape(1, D))


def kernel(h, edge_index, W_pool1, b_pool1, W_self1, W_neigh1, b1,
           W_pool2, b_pool2, W_self2, W_neigh2, b2):
    src = edge_index[0]
    dst = edge_index[1]

    p1 = _pool_tc(h, W_pool1, b_pool1)
    agg1 = _segmax(p1, src, dst, _LOTAB)[:N]
    h1, p2 = _combine_pool_tc(h, agg1, W_self1, W_neigh1, b1, W_pool2, b_pool2)
    agg2 = _segmax(p2, src, dst, _LOTAB)[:N]
    h2 = _combine_tc(h1, agg2, W_self2, W_neigh2, b2)
    return h2


# speedup vs baseline: 2.0599x; 2.0599x over previous
import functools

import jax
import jax.numpy as jnp
import numpy as np
from jax import lax
from jax.experimental import pallas as pl
from jax.experimental.pallas import tpu as pltpu
from jax.experimental.pallas import tpu_sc as plsc

N = 10000
E = 320000
D = 128

_INFO = plsc.get_sparse_core_info()
NC = _INFO.num_cores
NS = _INFO.num_subcores
NW = NC * NS
ROWS_PER_TILE = 320
NPAD = NW * ROWS_PER_TILE
CHUNK = 6400
NCHUNKS = E // CHUNK
GSUB = 128
LANES = 16
FB = D // LANES
DUMMY = ROWS_PER_TILE


UNROLL = 5
CPAD = CHUNK + GSUB


def _edge_copy(ei_hbm, eib, b, k, sem):
    return pltpu.make_async_copy(
        ei_hbm.at[pl.ds(0, 2), pl.ds(k * CHUNK, CHUNK)], eib.at[b], sem)


def _gather_copy(p_hbm, csrc, grows, g, bsel, sem):
    return pltpu.make_async_copy(
        p_hbm.at[csrc.at[pl.ds(g * GSUB, GSUB)]], grows.at[bsel], sem)


def _segmax_body(p_hbm, ei_hbm, lotab_hbm, agg_hbm,
                 acc, eib, csrc, cdst, grows, lov_v,
                 sem_e0, sem_e1, sem_g0, sem_g1):
    wid = lax.axis_index("s") * NC + lax.axis_index("c")
    lo = wid * ROWS_PER_TILE
    sem_e = [sem_e0, sem_e1]
    sem_g = [sem_g0, sem_g1]

    zf = jnp.zeros((LANES,), jnp.float32)
    zi = jnp.zeros((LANES,), jnp.int32)
    one = jnp.ones((LANES,), jnp.int32)
    dumv = jnp.full((LANES,), DUMMY, jnp.int32)

    pltpu.sync_copy(lotab_hbm.at[wid], lov_v)
    lov = lov_v[...]

    for b in range(2):
        _edge_copy(ei_hbm, eib, b, b, sem_e[b]).start()

    def zero_row(r, carry):
        for j in range(FB):
            acc[r, pl.ds(j * LANES, LANES)] = zf
        return carry
    lax.fori_loop(0, ROWS_PER_TILE + 1, zero_row, 0)

    def zero_c(i, c):
        csrc[pl.ds(i * LANES, LANES)] = zi
        cdst[pl.ds(i * LANES, LANES)] = dumv
        return c
    lax.fori_loop(0, CPAD // LANES, zero_c, 0)

    iota = lax.iota(jnp.int32, LANES)

    def chunk_pair(kk, carry):
        k0 = kk * 2
        for b in range(2):
            k = k0 + b
            _edge_copy(ei_hbm, eib, b, k, sem_e[b]).wait()

            def grp(i, wp):
                for u in range(UNROLL):
                    q = (i * UNROLL + u) * LANES
                    s16 = eib[b, 0, pl.ds(q, LANES)]
                    d16 = eib[b, 1, pl.ds(q, LANES)]
                    dl = d16 - lov
                    m = (dl >= 0) & (dl < ROWS_PER_TILE)
                    mi = jnp.where(m, one, zi)
                    pos = wp + plsc.cumsum(mi) - 1
                    plsc.store_scatter(csrc, [pos], s16, mask=m)
                    plsc.store_scatter(cdst, [pos], dl, mask=m)
                    wp = wp + plsc.all_reduce_population_count(m)
                return wp
            wpv = lax.fori_loop(0, CHUNK // LANES // UNROLL, grp,
                                jnp.zeros((LANES,), jnp.int32))
            cnt = jnp.max(wpv)

            pad = wpv + iota
            plsc.store_scatter(csrc, [pad], zi)
            plsc.store_scatter(cdst, [pad], dumv)

            @pl.when(k + 2 < NCHUNKS)
            def _():
                _edge_copy(ei_hbm, eib, b, k + 2, sem_e[b]).start()

            ngs = (cnt + GSUB - 1) // GSUB

            @pl.when(ngs > 0)
            def _():
                _gather_copy(p_hbm, csrc, grows, 0, 0, sem_g[0]).start()

            @pl.when(ngs > 1)
            def _():
                _gather_copy(p_hbm, csrc, grows, 1, 1, sem_g[1]).start()

            def gsub2(gg, c):
                for b2 in range(2):
                    g = gg * 2 + b2

                    @pl.when(g < ngs)
                    def _():
                        base = g * GSUB
                        _gather_copy(p_hbm, csrc, grows, g, b2,
                                     sem_g[b2]).wait()
                        ne = jnp.minimum(GSUB, cnt - base)
                        ngrp = (ne + LANES - 1) // LANES

                        def egrp(i, c2):
                            e0 = i * LANES
                            dl16 = cdst[pl.ds(base + e0, LANES)]
                            for l in range(LANES):
                                d = dl16[l]
                                for j in range(FB):
                                    sl = pl.ds(j * LANES, LANES)
                                    acc[d, sl] = jnp.maximum(
                                        acc[d, sl], grows[b2, e0 + l, sl])
                            return c2
                        lax.fori_loop(0, ngrp, egrp, 0)

                        @pl.when(g + 2 < ngs)
                        def _():
                            _gather_copy(p_hbm, csrc, grows, g + 2, b2,
                                         sem_g[b2]).start()
                return c
            lax.fori_loop(0, (ngs + 1) // 2, gsub2, 0)
        return carry

    lax.fori_loop(0, NCHUNKS // 2, chunk_pair, 0)
    pltpu.sync_copy(acc.at[pl.ds(0, ROWS_PER_TILE)],
                    agg_hbm.at[pl.ds(lo, ROWS_PER_TILE)])


_segmax = functools.partial(
    pl.kernel,
    out_type=jax.ShapeDtypeStruct((NPAD, D), jnp.float32),
    mesh=plsc.VectorSubcoreMesh(core_axis_name="c", subcore_axis_name="s"),
    compiler_params=pltpu.CompilerParams(needs_layout_passes=False),
    scratch_types=[
        pltpu.VMEM((ROWS_PER_TILE + 1, D), jnp.float32),
        pltpu.VMEM((2, 2, CHUNK), jnp.int32),
        pltpu.VMEM((CPAD,), jnp.int32),
        pltpu.VMEM((CPAD,), jnp.int32),
        pltpu.VMEM((2, GSUB, D), jnp.float32),
        pltpu.VMEM((LANES,), jnp.int32),
        pltpu.SemaphoreType.DMA,
        pltpu.SemaphoreType.DMA,
        pltpu.SemaphoreType.DMA,
        pltpu.SemaphoreType.DMA,
    ],
)(_segmax_body)

_LOTAB = np.tile(
    (np.arange(NW, dtype=np.int32) * ROWS_PER_TILE)[:, None], (1, LANES))


BLK = 1000


def _pool_tc_body(x_ref, w_ref, b_ref, p_ref):
    p_ref[...] = jnp.maximum(
        jnp.dot(x_ref[...], w_ref[...], preferred_element_type=jnp.float32)
        + b_ref[...], 0.0)


def _combine_pool_tc_body(x_ref, agg_ref, ws_ref, wn_ref, b_ref, wp_ref,
                          bp_ref, h_ref, p_ref):
    h = jnp.maximum(
        jnp.dot(x_ref[...], ws_ref[...], preferred_element_type=jnp.float32)
        + jnp.dot(agg_ref[...], wn_ref[...], preferred_element_type=jnp.float32)
        + b_ref[...], 0.0)
    h_ref[...] = h
    p_ref[...] = jnp.maximum(
        jnp.dot(h, wp_ref[...], preferred_element_type=jnp.float32)
        + bp_ref[...], 0.0)


def _combine_tc_body(x_ref, agg_ref, ws_ref, wn_ref, b_ref, h_ref):
    h_ref[...] = jnp.maximum(
        jnp.dot(x_ref[...], ws_ref[...], preferred_element_type=jnp.float32)
        + jnp.dot(agg_ref[...], wn_ref[...], preferred_element_type=jnp.float32)
        + b_ref[...], 0.0)


def _row_spec():
    return pl.BlockSpec((BLK, D), lambda i: (i, 0))


def _full_spec():
    return pl.BlockSpec((D, D), lambda i: (0, 0))


def _bias_spec():
    return pl.BlockSpec((1, D), lambda i: (0, 0))


def _pool_tc(x, w, b):
    return pl.pallas_call(
        _pool_tc_body,
        grid=(N // BLK,),
        in_specs=[_row_spec(), _full_spec(), _bias_spec()],
        out_specs=_row_spec(),
        out_shape=jax.ShapeDtypeStruct((N, D), jnp.float32),
    )(x, w, b.reshape(1, D))


def _combine_pool_tc(x, agg, ws, wn, b, wp, bp):
    return pl.pallas_call(
        _combine_pool_tc_body,
        grid=(N // BLK,),
        in_specs=[_row_spec(), _row_spec(), _full_spec(), _full_spec(),
                  _bias_spec(), _full_spec(), _bias_spec()],
        out_specs=[_row_spec(), _row_spec()],
        out_shape=[jax.ShapeDtypeStruct((N, D), jnp.float32),
                   jax.ShapeDtypeStruct((N, D), jnp.float32)],
    )(x, agg, ws, wn, b.reshape(1, D), wp, bp.reshape(1, D))


def _combine_tc(x, agg, ws, wn, b):
    return pl.pallas_call(
        _combine_tc_body,
        grid=(N // BLK,),
        in_specs=[_row_spec(), _row_spec(), _full_spec(), _full_spec(),
                  _bias_spec()],
        out_specs=_row_spec(),
        out_shape=jax.ShapeDtypeStruct((N, D), jnp.float32),
    )(x, agg, ws, wn, b.reshape(1, D))


def kernel(h, edge_index, W_pool1, b_pool1, W_self1, W_neigh1, b1,
           W_pool2, b_pool2, W_self2, W_neigh2, b2):
    p1 = _pool_tc(h, W_pool1, b_pool1)
    agg1 = _segmax(p1, edge_index, _LOTAB)[:N]
    h1, p2 = _combine_pool_tc(h, agg1, W_self1, W_neigh1, b1, W_pool2, b_pool2)
    agg2 = _segmax(p2, edge_index, _LOTAB)[:N]
    h2 = _combine_tc(h1, agg2, W_self2, W_neigh2, b2)
    return h2
